# speedup vs baseline: 1.6314x; 1.6314x over previous
import jax
import jax.numpy as jnp
from jax.experimental import pallas as pl
from jax.experimental.pallas import tpu as pltpu

NUM_EXPERTS = 64
TOP_K = 2
BT = 1024


def _router_kernel(x_ref, w_ref, logits_ref, weights_ref, experts_ref):
    x = x_ref[...]
    w = w_ref[...]
    logits = jnp.dot(x, w, preferred_element_type=jnp.float32)
    logits_ref[...] = logits

    iota = jax.lax.broadcasted_iota(jnp.int32, logits.shape, 1)
    neg_inf = jnp.float32(float("-inf"))

    m0 = jnp.max(logits, axis=-1, keepdims=True)
    i0 = jnp.min(jnp.where(logits == m0, iota, NUM_EXPERTS), axis=-1,
                 keepdims=True)
    masked = jnp.where(iota == i0, neg_inf, logits)
    m1 = jnp.max(masked, axis=-1, keepdims=True)
    i1 = jnp.min(jnp.where(masked == m1, iota, NUM_EXPERTS), axis=-1,
                 keepdims=True)

    e = jnp.exp(m1 - m0)
    s = 1.0 / (1.0 + e)
    weights_ref[...] = jnp.concatenate([s, e * s], axis=-1)
    experts_ref[...] = jnp.concatenate([i0, i1], axis=-1)


@jax.jit
def kernel(hidden_states, gate_w):
    b, seq, hidden = hidden_states.shape
    n_tok = b * seq
    x = hidden_states.reshape(n_tok, hidden)
    w = gate_w.T

    grid = (n_tok // BT,)
    logits, weights, experts = pl.pallas_call(
        _router_kernel,
        grid=grid,
        in_specs=[
            pl.BlockSpec((BT, hidden), lambda i: (i, 0)),
            pl.BlockSpec((hidden, NUM_EXPERTS), lambda i: (0, 0)),
        ],
        out_specs=[
            pl.BlockSpec((BT, NUM_EXPERTS), lambda i: (i, 0)),
            pl.BlockSpec((BT, TOP_K), lambda i: (i, 0)),
            pl.BlockSpec((BT, TOP_K), lambda i: (i, 0)),
        ],
        out_shape=[
            jax.ShapeDtypeStruct((n_tok, NUM_EXPERTS), jnp.float32),
            jax.ShapeDtypeStruct((n_tok, TOP_K), jnp.float32),
            jax.ShapeDtypeStruct((n_tok, TOP_K), jnp.int32),
        ],
        compiler_params=pltpu.CompilerParams(
            dimension_semantics=("arbitrary",),
        ),
    )(x, w)

    return (
        weights.reshape(b, seq, TOP_K),
        experts.reshape(b, seq, TOP_K),
        logits.reshape(b, seq, NUM_EXPERTS),
    )


# speedup vs baseline: 1.6643x; 1.0202x over previous
import jax
import jax.numpy as jnp
from jax.experimental import pallas as pl
from jax.experimental.pallas import tpu as pltpu

NUM_EXPERTS = 64
TOP_K = 2
BT = 1024


def _router_kernel(x_ref, w_ref, logits_ref, weights_ref, experts_ref):
    x = x_ref[...]
    w = w_ref[...]
    logits = jax.lax.dot_general(
        x, w, (((1,), (1,)), ((), ())),
        preferred_element_type=jnp.float32)
    logits_ref[...] = logits

    iota = jax.lax.broadcasted_iota(jnp.int32, logits.shape, 1)
    neg_inf = jnp.float32(float("-inf"))

    m0 = jnp.max(logits, axis=-1, keepdims=True)
    i0 = jnp.min(jnp.where(logits == m0, iota, NUM_EXPERTS), axis=-1,
                 keepdims=True)
    masked = jnp.where(iota == i0, neg_inf, logits)
    m1 = jnp.max(masked, axis=-1, keepdims=True)
    i1 = jnp.min(jnp.where(masked == m1, iota, NUM_EXPERTS), axis=-1,
                 keepdims=True)

    e = jnp.exp(m1 - m0)
    s = 1.0 / (1.0 + e)
    weights_ref[...] = jnp.concatenate([s, e * s], axis=-1)
    experts_ref[...] = jnp.concatenate([i0, i1], axis=-1)


@jax.jit
def kernel(hidden_states, gate_w):
    b, seq, hidden = hidden_states.shape
    n_tok = b * seq
    x = hidden_states.reshape(n_tok, hidden)

    grid = (n_tok // BT,)
    logits, weights, experts = pl.pallas_call(
        _router_kernel,
        grid=grid,
        in_specs=[
            pl.BlockSpec((BT, hidden), lambda i: (i, 0)),
            pl.BlockSpec((NUM_EXPERTS, hidden), lambda i: (0, 0)),
        ],
        out_specs=[
            pl.BlockSpec((BT, NUM_EXPERTS), lambda i: (i, 0)),
            pl.BlockSpec((BT, TOP_K), lambda i: (i, 0)),
            pl.BlockSpec((BT, TOP_K), lambda i: (i, 0)),
        ],
        out_shape=[
            jax.ShapeDtypeStruct((n_tok, NUM_EXPERTS), jnp.float32),
            jax.ShapeDtypeStruct((n_tok, TOP_K), jnp.float32),
            jax.ShapeDtypeStruct((n_tok, TOP_K), jnp.int32),
        ],
        compiler_params=pltpu.CompilerParams(
            dimension_semantics=("arbitrary",),
        ),
    )(x, gate_w)

    return (
        weights.reshape(b, seq, TOP_K),
        experts.reshape(b, seq, TOP_K),
        logits.reshape(b, seq, NUM_EXPERTS),
    )


# speedup vs baseline: 1.8234x; 1.0956x over previous
import jax
import jax.numpy as jnp
from jax.experimental import pallas as pl
from jax.experimental.pallas import tpu as pltpu

NUM_EXPERTS = 64
TOP_K = 2
BT = 2048


def _router_kernel(x_ref, w_ref, logits_ref, weights_ref, experts_ref):
    x = x_ref[...]
    w = w_ref[...]
    logits = jax.lax.dot_general(
        x, w, (((1,), (1,)), ((), ())),
        preferred_element_type=jnp.float32)
    logits_ref[...] = logits

    iota = jax.lax.broadcasted_iota(jnp.int32, logits.shape, 1)
    neg_inf = jnp.float32(float("-inf"))

    m0 = jnp.max(logits, axis=-1, keepdims=True)
    i0 = jnp.min(jnp.where(logits == m0, iota, NUM_EXPERTS), axis=-1,
                 keepdims=True)
    masked = jnp.where(iota == i0, neg_inf, logits)
    m1 = jnp.max(masked, axis=-1, keepdims=True)
    i1 = jnp.min(jnp.where(masked == m1, iota, NUM_EXPERTS), axis=-1,
                 keepdims=True)

    e = jnp.exp(m1 - m0)
    s = 1.0 / (1.0 + e)
    weights_ref[...] = jnp.concatenate([s, e * s], axis=-1)
    experts_ref[...] = jnp.concatenate([i0, i1], axis=-1)


@jax.jit
def kernel(hidden_states, gate_w):
    b, seq, hidden = hidden_states.shape
    n_tok = b * seq
    x = hidden_states.reshape(n_tok, hidden)

    grid = (n_tok // BT,)
    logits, weights, experts = pl.pallas_call(
        _router_kernel,
        grid=grid,
        in_specs=[
            pl.BlockSpec((BT, hidden), lambda i: (i, 0)),
            pl.BlockSpec((NUM_EXPERTS, hidden), lambda i: (0, 0)),
        ],
        out_specs=[
            pl.BlockSpec((BT, NUM_EXPERTS), lambda i: (i, 0)),
            pl.BlockSpec((BT, TOP_K), lambda i: (i, 0)),
            pl.BlockSpec((BT, TOP_K), lambda i: (i, 0)),
        ],
        out_shape=[
            jax.ShapeDtypeStruct((n_tok, NUM_EXPERTS), jnp.float32),
            jax.ShapeDtypeStruct((n_tok, TOP_K), jnp.float32),
            jax.ShapeDtypeStruct((n_tok, TOP_K), jnp.int32),
        ],
        compiler_params=pltpu.CompilerParams(
            dimension_semantics=("arbitrary",),
        ),
    )(x, gate_w)

    return (
        weights.reshape(b, seq, TOP_K),
        experts.reshape(b, seq, TOP_K),
        logits.reshape(b, seq, NUM_EXPERTS),
    )


# speedup vs baseline: 1.9200x; 1.0530x over previous
import jax
import jax.numpy as jnp
from jax.experimental import pallas as pl
from jax.experimental.pallas import tpu as pltpu

NUM_EXPERTS = 64
TOP_K = 2
BT = 4096


def _router_kernel(x_ref, w_ref, logits_ref, weights_ref, experts_ref):
    x = x_ref[...]
    w = w_ref[...]
    logits = jax.lax.dot_general(
        x, w, (((1,), (1,)), ((), ())),
        preferred_element_type=jnp.float32)
    logits_ref[...] = logits

    iota = jax.lax.broadcasted_iota(jnp.int32, logits.shape, 1)
    neg_inf = jnp.float32(float("-inf"))

    m0 = jnp.max(logits, axis=-1, keepdims=True)
    i0 = jnp.min(jnp.where(logits == m0, iota, NUM_EXPERTS), axis=-1,
                 keepdims=True)
    masked = jnp.where(iota == i0, neg_inf, logits)
    m1 = jnp.max(masked, axis=-1, keepdims=True)
    i1 = jnp.min(jnp.where(masked == m1, iota, NUM_EXPERTS), axis=-1,
                 keepdims=True)

    e = jnp.exp(m1 - m0)
    s = 1.0 / (1.0 + e)
    weights_ref[...] = jnp.concatenate([s, e * s], axis=-1)
    experts_ref[...] = jnp.concatenate([i0, i1], axis=-1)


@jax.jit
def kernel(hidden_states, gate_w):
    b, seq, hidden = hidden_states.shape
    n_tok = b * seq
    x = hidden_states.reshape(n_tok, hidden)

    grid = (n_tok // BT,)
    logits, weights, experts = pl.pallas_call(
        _router_kernel,
        grid=grid,
        in_specs=[
            pl.BlockSpec((BT, hidden), lambda i: (i, 0)),
            pl.BlockSpec((NUM_EXPERTS, hidden), lambda i: (0, 0)),
        ],
        out_specs=[
            pl.BlockSpec((BT, NUM_EXPERTS), lambda i: (i, 0)),
            pl.BlockSpec((BT, TOP_K), lambda i: (i, 0)),
            pl.BlockSpec((BT, TOP_K), lambda i: (i, 0)),
        ],
        out_shape=[
            jax.ShapeDtypeStruct((n_tok, NUM_EXPERTS), jnp.float32),
            jax.ShapeDtypeStruct((n_tok, TOP_K), jnp.float32),
            jax.ShapeDtypeStruct((n_tok, TOP_K), jnp.int32),
        ],
        compiler_params=pltpu.CompilerParams(
            dimension_semantics=("arbitrary",),
        ),
    )(x, gate_w)

    return (
        weights.reshape(b, seq, TOP_K),
        experts.reshape(b, seq, TOP_K),
        logits.reshape(b, seq, NUM_EXPERTS),
    )


# speedup vs baseline: 2.0580x; 1.0719x over previous
import jax
import jax.numpy as jnp
from jax.experimental import pallas as pl
from jax.experimental.pallas import tpu as pltpu

NUM_EXPERTS = 64
TOP_K = 2
BT = 4096


def _router_kernel(x_ref, w_ref, logits_ref, weights_ref, experts_ref):
    x = x_ref[0]
    w = w_ref[...]
    logits = jax.lax.dot_general(
        x, w, (((1,), (1,)), ((), ())),
        preferred_element_type=jnp.float32)
    logits_ref[0] = logits

    iota = jax.lax.broadcasted_iota(jnp.int32, logits.shape, 1)
    neg_inf = jnp.float32(float("-inf"))

    m0 = jnp.max(logits, axis=-1, keepdims=True)
    i0 = jnp.min(jnp.where(logits == m0, iota, NUM_EXPERTS), axis=-1,
                 keepdims=True)
    masked = jnp.where(iota == i0, neg_inf, logits)
    m1 = jnp.max(masked, axis=-1, keepdims=True)
    i1 = jnp.min(jnp.where(masked == m1, iota, NUM_EXPERTS), axis=-1,
                 keepdims=True)

    e = jnp.exp(m1 - m0)
    s = 1.0 / (1.0 + e)
    weights_ref[0] = jnp.concatenate([s, e * s], axis=-1)
    experts_ref[0] = jnp.concatenate([i0, i1], axis=-1)


@jax.jit
def kernel(hidden_states, gate_w):
    b, seq, hidden = hidden_states.shape
    sb = seq // BT

    grid = (b * sb,)
    logits, weights, experts = pl.pallas_call(
        _router_kernel,
        grid=grid,
        in_specs=[
            pl.BlockSpec((1, BT, hidden), lambda i: (i // sb, i % sb, 0)),
            pl.BlockSpec((NUM_EXPERTS, hidden), lambda i: (0, 0)),
        ],
        out_specs=[
            pl.BlockSpec((1, BT, NUM_EXPERTS), lambda i: (i // sb, i % sb, 0)),
            pl.BlockSpec((1, BT, TOP_K), lambda i: (i // sb, i % sb, 0)),
            pl.BlockSpec((1, BT, TOP_K), lambda i: (i // sb, i % sb, 0)),
        ],
        out_shape=[
            jax.ShapeDtypeStruct((b, seq, NUM_EXPERTS), jnp.float32),
            jax.ShapeDtypeStruct((b, seq, TOP_K), jnp.float32),
            jax.ShapeDtypeStruct((b, seq, TOP_K), jnp.int32),
        ],
        compiler_params=pltpu.CompilerParams(
            dimension_semantics=("arbitrary",),
        ),
    )(hidden_states, gate_w)

    return weights, experts, logits


# speedup vs baseline: 3.0759x; 1.4946x over previous
import jax
import jax.numpy as jnp
from jax.experimental import pallas as pl
from jax.experimental.pallas import tpu as pltpu

NUM_EXPERTS = 64
TOP_K = 2
BT = 4096


def _router_kernel(x_ref, w_ref, logits_ref, wt_ref, et_ref):
    x = x_ref[0]
    w = w_ref[...]
    logits = jax.lax.dot_general(
        x, w, (((1,), (1,)), ((), ())),
        preferred_element_type=jnp.float32)
    logits_ref[0] = logits

    logits_t = jax.lax.dot_general(
        w, x, (((1,), (1,)), ((), ())),
        preferred_element_type=jnp.float32)

    iota = jax.lax.broadcasted_iota(jnp.int32, logits_t.shape, 0)
    neg_inf = jnp.float32(float("-inf"))

    m0 = jnp.max(logits_t, axis=0, keepdims=True)
    i0 = jnp.min(jnp.where(logits_t == m0, iota, NUM_EXPERTS), axis=0,
                 keepdims=True)
    masked = jnp.where(iota == i0, neg_inf, logits_t)
    m1 = jnp.max(masked, axis=0, keepdims=True)
    i1 = jnp.min(jnp.where(masked == m1, iota, NUM_EXPERTS), axis=0,
                 keepdims=True)

    e = jnp.exp(m1 - m0)
    s = 1.0 / (1.0 + e)
    wt_ref[0] = jnp.concatenate([s, e * s], axis=0)
    et_ref[0] = jnp.concatenate([i0, i1], axis=0)


def kernel(hidden_states, gate_w):
    b, seq, hidden = hidden_states.shape
    sb = seq // BT

    grid = (b * sb,)
    logits, weights_t, experts_t = pl.pallas_call(
        _router_kernel,
        grid=grid,
        in_specs=[
            pl.BlockSpec((1, BT, hidden), lambda i: (i // sb, i % sb, 0)),
            pl.BlockSpec((NUM_EXPERTS, hidden), lambda i: (0, 0)),
        ],
        out_specs=[
            pl.BlockSpec((1, BT, NUM_EXPERTS), lambda i: (i // sb, i % sb, 0)),
            pl.BlockSpec((1, TOP_K, BT), lambda i: (i // sb, 0, i % sb)),
            pl.BlockSpec((1, TOP_K, BT), lambda i: (i // sb, 0, i % sb)),
        ],
        out_shape=[
            jax.ShapeDtypeStruct((b, seq, NUM_EXPERTS), jnp.float32),
            jax.ShapeDtypeStruct((b, TOP_K, seq), jnp.float32),
            jax.ShapeDtypeStruct((b, TOP_K, seq), jnp.int32),
        ],
        compiler_params=pltpu.CompilerParams(
            dimension_semantics=("arbitrary",),
        ),
    )(hidden_states, gate_w)

    return weights_t.swapaxes(1, 2), experts_t.swapaxes(1, 2), logits


# speedup vs baseline: 3.9725x; 1.2915x over previous
import jax
import jax.numpy as jnp
from jax.experimental import pallas as pl
from jax.experimental.pallas import tpu as pltpu
from jax.experimental.layout import Layout, with_layout_constraint

NUM_EXPERTS = 64
TOP_K = 2
BT = 4096


def _router_kernel(x_ref, w_ref, logits_ref, wt_ref, et_ref):
    x = x_ref[0]
    w = w_ref[...]
    logits = jax.lax.dot_general(
        x, w, (((1,), (1,)), ((), ())),
        preferred_element_type=jnp.float32)
    logits_ref[0] = logits

    logits_t = jax.lax.dot_general(
        w, x, (((1,), (1,)), ((), ())),
        preferred_element_type=jnp.float32)

    iota = jax.lax.broadcasted_iota(jnp.int32, logits_t.shape, 0)
    neg_inf = jnp.float32(float("-inf"))

    m0 = jnp.max(logits_t, axis=0, keepdims=True)
    i0 = jnp.min(jnp.where(logits_t == m0, iota, NUM_EXPERTS), axis=0,
                 keepdims=True)
    masked = jnp.where(iota == i0, neg_inf, logits_t)
    m1 = jnp.max(masked, axis=0, keepdims=True)
    i1 = jnp.min(jnp.where(masked == m1, iota, NUM_EXPERTS), axis=0,
                 keepdims=True)

    e = jnp.exp(m1 - m0)
    s = 1.0 / (1.0 + e)
    wt_ref[0] = jnp.concatenate([s, e * s], axis=0)
    et_ref[0] = jnp.concatenate([i0, i1], axis=0)


def kernel(hidden_states, gate_w):
    b, seq, hidden = hidden_states.shape
    sb = seq // BT

    grid = (b * sb,)
    logits, weights_t, experts_t = pl.pallas_call(
        _router_kernel,
        grid=grid,
        in_specs=[
            pl.BlockSpec((1, BT, hidden), lambda i: (i // sb, i % sb, 0)),
            pl.BlockSpec((NUM_EXPERTS, hidden), lambda i: (0, 0)),
        ],
        out_specs=[
            pl.BlockSpec((1, BT, NUM_EXPERTS), lambda i: (i // sb, i % sb, 0)),
            pl.BlockSpec((1, TOP_K, BT), lambda i: (i // sb, 0, i % sb)),
            pl.BlockSpec((1, TOP_K, BT), lambda i: (i // sb, 0, i % sb)),
        ],
        out_shape=[
            jax.ShapeDtypeStruct((b, seq, NUM_EXPERTS), jnp.float32),
            jax.ShapeDtypeStruct((b, TOP_K, seq), jnp.float32),
            jax.ShapeDtypeStruct((b, TOP_K, seq), jnp.int32),
        ],
        compiler_params=pltpu.CompilerParams(
            dimension_semantics=("arbitrary",),
        ),
    )(hidden_states, gate_w)

    logits = with_layout_constraint(
        logits, Layout((0, 1, 2), ((8, 128),)))
    return weights_t.swapaxes(1, 2), experts_t.swapaxes(1, 2), logits
